# baseline (speedup 1.0000x reference)
import jax
import jax.numpy as jnp
from jax.experimental import pallas as pl
from jax.experimental.pallas import tpu as pltpu

_CH = 8


def _sig(x):
    return 0.5 * jnp.tanh(0.5 * x) + 0.5


def _lstm_kernel(x_hbm, wih_ref, whh_ref, bih_ref, bhh_ref, wout_ref,
                 bout_ref, out_ref, xbuf, sem, wih_s, whh_s, b_s, wout_s):
    seq = x_hbm.shape[0]
    nbuf, ch, B, I = xbuf.shape
    fourH = whh_ref.shape[0]
    H = whh_ref.shape[1]
    n_out = wout_ref.shape[0]
    nchunk = seq // ch

    def chunk_copy(jj):
        return pltpu.make_async_copy(
            x_hbm.at[pl.ds(jj * ch, ch)], xbuf.at[jj % nbuf],
            sem.at[jj % nbuf])

    chunk_copy(0).start()

    wih_s[...] = wih_ref[...].T.astype(jnp.bfloat16)
    whh_s[...] = whh_ref[...].T
    b_s[...] = bih_ref[...] + bhh_ref[...]
    wout_s[...] = jnp.zeros_like(wout_s)
    wout_s[:, :n_out] = wout_ref[...].T

    wih = wih_s[...]
    whh = whh_s[...]
    b = b_s[...]

    h = jnp.zeros((B, H), jnp.float32)
    c = jnp.zeros((B, H), jnp.float32)

    for jj in range(nchunk):
        if jj + 1 < nchunk:
            chunk_copy(jj + 1).start()
        chunk_copy(jj).wait()
        xc = xbuf[jj % nbuf]

        def proj(t):
            return jnp.dot(xc[t].astype(jnp.bfloat16), wih,
                           preferred_element_type=jnp.float32) + b

        gx_t = proj(0)
        for t in range(ch):
            gates = gx_t + jnp.dot(h, whh,
                                   preferred_element_type=jnp.float32)
            if t + 1 < ch:
                gx_t = proj(t + 1)
            i_g = _sig(gates[:, 0 * H:1 * H])
            f_g = _sig(gates[:, 1 * H:2 * H])
            g_g = jnp.tanh(gates[:, 2 * H:3 * H])
            o_g = _sig(gates[:, 3 * H:4 * H])
            c = f_g * c + i_g * g_g
            h = o_g * jnp.tanh(c)

    out_ref[...] = (
        jnp.dot(h, wout_s[...], preferred_element_type=jnp.float32)
        + bout_ref[...]
    ).astype(out_ref.dtype)


def kernel(x, w_ih, w_hh, b_ih, b_hh, w_out, b_out):
    seq, B, I = x.shape
    fourH, H = w_hh.shape
    n_out = w_out.shape[0]
    n_out_pad = ((n_out + 127) // 128) * 128

    x = x.astype(jnp.float32)
    bih2 = b_ih.reshape(1, fourH).astype(jnp.float32)
    bhh2 = b_hh.reshape(1, fourH).astype(jnp.float32)
    if n_out == n_out_pad:
        bout2 = b_out.reshape(1, n_out).astype(jnp.float32)
    else:
        bout2 = jnp.zeros((1, n_out_pad), jnp.float32).at[:, :n_out].set(
            b_out.reshape(1, n_out))

    out_pad = pl.pallas_call(
        _lstm_kernel,
        out_shape=jax.ShapeDtypeStruct((B, n_out_pad), jnp.float32),
        in_specs=[
            pl.BlockSpec(memory_space=pl.ANY),
            pl.BlockSpec((fourH, I), lambda: (0, 0)),
            pl.BlockSpec((fourH, H), lambda: (0, 0)),
            pl.BlockSpec((1, fourH), lambda: (0, 0)),
            pl.BlockSpec((1, fourH), lambda: (0, 0)),
            pl.BlockSpec((n_out, H), lambda: (0, 0)),
            pl.BlockSpec((1, n_out_pad), lambda: (0, 0)),
        ],
        out_specs=pl.BlockSpec((B, n_out_pad), lambda: (0, 0)),
        scratch_shapes=[
            pltpu.VMEM((2, _CH, B, I), jnp.float32),
            pltpu.SemaphoreType.DMA((2,)),
            pltpu.VMEM((I, fourH), jnp.bfloat16),
            pltpu.VMEM((H, fourH), jnp.float32),
            pltpu.VMEM((1, fourH), jnp.float32),
            pltpu.VMEM((H, n_out_pad), jnp.float32),
        ],
    )(x, w_ih, w_hh, bih2, bhh2, w_out, bout2)

    return out_pad[:, :n_out].astype(x.dtype)


# speedup vs baseline: 1.0740x; 1.0740x over previous
import jax
import jax.numpy as jnp
from jax.experimental import pallas as pl
from jax.experimental.pallas import tpu as pltpu

_NCHUNK = 4


def _sig(x):
    return 0.5 * jnp.tanh(0.5 * x) + 0.5


def _lstm_kernel(x_ref, wih_ref, whh_ref, bih_ref, bhh_ref, wout_ref,
                 bout_ref, out_ref, h_ref, c_ref, wih_s, whh_s, b_s, wout_s):
    ch, B, I = x_ref.shape
    fourH = whh_ref.shape[0]
    H = whh_ref.shape[1]
    n_out = wout_ref.shape[0]
    n_out_pad = wout_s.shape[1]
    j = pl.program_id(0)
    nchunk = pl.num_programs(0)

    @pl.when(j == 0)
    def _init():
        h_ref[...] = jnp.zeros_like(h_ref)
        c_ref[...] = jnp.zeros_like(c_ref)
        wih_s[...] = wih_ref[...].T.astype(jnp.bfloat16)
        whh_s[...] = whh_ref[...].T
        b_s[...] = bih_ref[...] + bhh_ref[...]
        wout_s[...] = jnp.zeros_like(wout_s)
        wout_s[:, :n_out] = wout_ref[...].T

    wih = wih_s[...]
    whh = whh_s[...]
    b = b_s[...]

    def proj(t):
        return jnp.dot(x_ref[t].astype(jnp.bfloat16), wih,
                       preferred_element_type=jnp.float32) + b

    h = h_ref[...]
    c = c_ref[...]
    gx_t = proj(0)
    for t in range(ch):
        gates = gx_t + jnp.dot(h, whh, preferred_element_type=jnp.float32)
        if t + 1 < ch:
            gx_t = proj(t + 1)
        i_g = _sig(gates[:, 0 * H:1 * H])
        f_g = _sig(gates[:, 1 * H:2 * H])
        g_g = jnp.tanh(gates[:, 2 * H:3 * H])
        o_g = _sig(gates[:, 3 * H:4 * H])
        c = f_g * c + i_g * g_g
        h = o_g * jnp.tanh(c)
    h_ref[...] = h
    c_ref[...] = c

    @pl.when(j == nchunk - 1)
    def _finish():
        out_ref[...] = (
            jnp.dot(h, wout_s[...], preferred_element_type=jnp.float32)
            + bout_ref[...]
        ).astype(out_ref.dtype)


def kernel(x, w_ih, w_hh, b_ih, b_hh, w_out, b_out):
    seq, B, I = x.shape
    fourH, H = w_hh.shape
    n_out = w_out.shape[0]
    n_out_pad = ((n_out + 127) // 128) * 128
    ch = seq // _NCHUNK

    x = x.astype(jnp.float32)
    bih2 = b_ih.reshape(1, fourH).astype(jnp.float32)
    bhh2 = b_hh.reshape(1, fourH).astype(jnp.float32)
    if n_out == n_out_pad:
        bout2 = b_out.reshape(1, n_out).astype(jnp.float32)
    else:
        bout2 = jnp.zeros((1, n_out_pad), jnp.float32).at[:, :n_out].set(
            b_out.reshape(1, n_out))

    grid_spec = pltpu.PrefetchScalarGridSpec(
        num_scalar_prefetch=0,
        grid=(_NCHUNK,),
        in_specs=[
            pl.BlockSpec((ch, B, I), lambda j: (j, 0, 0)),
            pl.BlockSpec((fourH, I), lambda j: (0, 0)),
            pl.BlockSpec((fourH, H), lambda j: (0, 0)),
            pl.BlockSpec((1, fourH), lambda j: (0, 0)),
            pl.BlockSpec((1, fourH), lambda j: (0, 0)),
            pl.BlockSpec((n_out, H), lambda j: (0, 0)),
            pl.BlockSpec((1, n_out_pad), lambda j: (0, 0)),
        ],
        out_specs=pl.BlockSpec((B, n_out_pad), lambda j: (0, 0)),
        scratch_shapes=[
            pltpu.VMEM((B, H), jnp.float32),
            pltpu.VMEM((B, H), jnp.float32),
            pltpu.VMEM((I, fourH), jnp.bfloat16),
            pltpu.VMEM((H, fourH), jnp.float32),
            pltpu.VMEM((1, fourH), jnp.float32),
            pltpu.VMEM((H, n_out_pad), jnp.float32),
        ],
    )

    out_pad = pl.pallas_call(
        _lstm_kernel,
        out_shape=jax.ShapeDtypeStruct((B, n_out_pad), jnp.float32),
        grid_spec=grid_spec,
        compiler_params=pltpu.CompilerParams(
            dimension_semantics=("arbitrary",)),
    )(x, w_ih, w_hh, bih2, bhh2, w_out, bout2)

    return out_pad[:, :n_out].astype(x.dtype)
